# baseline (speedup 1.0000x reference)
import functools

import jax
import jax.numpy as jnp
from jax import lax
from jax.experimental import pallas as pl
from jax.experimental.pallas import tpu as pltpu
from jax.experimental.pallas import tpu_sc as plsc

_B = 4096
_L = 200
_EMB = 64
_V = 1000000
_NW = 32
_VP = _V // 2
_NBLK = (_V // 128)
_BPW = _NBLK // _NW
_REM = _NBLK - _BPW * _NW
_PAIRS_A = _BPW // 2
_BB = _B // _NW
_PAIRS_B = _L // 2

_mesh = plsc.VectorSubcoreMesh(core_axis_name="c", subcore_axis_name="s")
_params = pltpu.CompilerParams(
    use_tc_tiling_on_sc=True,
    needs_layout_passes=False,
    disable_bounds_checks=True,
)


def _iota16():
    return lax.iota(jnp.int32, 16)


@functools.partial(
    pl.kernel,
    mesh=_mesh,
    out_type=jax.ShapeDtypeStruct((_VP, 128), jnp.float32),
    scratch_types=[
        pltpu.VMEM((4, _EMB, 128), jnp.float32),
        pltpu.VMEM((4, _EMB, 128), jnp.float32),
        pltpu.VMEM((32, 128), jnp.float32),
        pltpu.SemaphoreType.DMA,
        pltpu.SemaphoreType.DMA,
    ],
    compiler_params=_params,
)
def _transpose_table(tT_hbm, tail_hbm, tlin_hbm, inv, obuf, tailv, isem, osem):
    wid = lax.axis_index("s") * 2 + lax.axis_index("c")
    base = wid * _BPW
    it16 = _iota16()

    def _start_in(j, b):
        pltpu.async_copy(tT_hbm.at[:, pl.ds(j * 128, 128)], inv.at[b], isem)

    def _wait_in(b):
        pltpu.make_async_copy(tT_hbm.at[:, pl.ds(0, 128)], inv.at[b], isem).wait()

    def _start_out(j, b):
        pltpu.async_copy(obuf.at[b], tlin_hbm.at[pl.ds(j * 64, 64)], osem)

    def _wait_out(b):
        pltpu.make_async_copy(obuf.at[b], tlin_hbm.at[pl.ds(0, 64)], osem).wait()

    zero16 = jnp.zeros((16,), jnp.int32)
    sbases = [((it16 + 16 * k) >> 1) * 128 + ((it16 + 16 * k) & 1) * 64
              for k in range(8)]

    def _vpu_transpose(b):
        @plsc.parallel_loop(0, 64, unroll=8)
        def cbody(c):
            for k in range(8):
                val = inv[b, c, pl.ds(16 * k, 16)]
                plsc.store_scatter(obuf.at[b], [zero16, sbases[k] + c], val)

    for b in range(4):
        _start_in(base + b, b)

    def body(u, carry):
        t0 = base + 4 * u
        for b in range(4):
            _wait_in(b)

            @pl.when(u > 0)
            def _(b=b):
                _wait_out(b)

            _vpu_transpose(b)
            _start_out(t0 + b, b)

            @pl.when(u + 1 < _BPW // 4)
            def _(t=t0 + b, b=b):
                _start_in(t + 4, b)
        return carry

    lax.fori_loop(0, _BPW // 4, body, 0)
    for b in range(4):
        _wait_out(b)

    @pl.when(wid < _REM)
    def _():
        jx = _NBLK - _REM + wid
        _start_in(jx, 0)
        _wait_in(0)
        _vpu_transpose(0)
        _start_out(jx, 0)
        _wait_out(0)

    @pl.when(wid == _NW - 1)
    def _():
        pltpu.sync_copy(tail_hbm, tailv)
        pltpu.sync_copy(tailv, tlin_hbm.at[pl.ds(_NBLK * 64, 32)])


@functools.partial(
    pl.kernel,
    mesh=_mesh,
    out_type=jax.ShapeDtypeStruct((_L, _EMB, _B), jnp.float32),
    scratch_types=[
        pltpu.VMEM((_L, 128), jnp.int32),
        pltpu.VMEM((4, 128), jnp.int32),
        pltpu.VMEM((4, 128, 128), jnp.float32),
        pltpu.VMEM((4, _EMB, 128), jnp.float32),
        pltpu.SemaphoreType.DMA,
        pltpu.SemaphoreType.DMA,
    ],
    compiler_params=_params,
)
def _emb_gather(seqT_hbm, tlin_hbm, out_hbm, idxv, pidx, gbuf, tbuf, gsem, osem):
    wid = lax.axis_index("s") * 2 + lax.axis_index("c")
    bcol = wid * _BB
    it16 = _iota16()

    pltpu.sync_copy(seqT_hbm.at[:, pl.ds(bcol, _BB)], idxv)

    def _make_pidx(l, b):
        for k in range(8):
            v = idxv[l, pl.ds(16 * k, 16)]
            pidx[b, pl.ds(16 * k, 16)] = lax.shift_right_logical(v, 1)

    def _start_gather(b):
        pltpu.async_copy(tlin_hbm.at[pidx.at[b]], gbuf.at[b], gsem)

    def _wait_gather(b):
        pltpu.make_async_copy(tlin_hbm.at[pidx.at[b]], gbuf.at[b], gsem).wait()

    def _start_out(l, b):
        pltpu.async_copy(tbuf.at[b], out_hbm.at[l, :, pl.ds(bcol, _BB)], osem)

    def _wait_out(b):
        pltpu.make_async_copy(tbuf.at[b], out_hbm.at[0, :, pl.ds(bcol, _BB)], osem).wait()

    zero16 = jnp.zeros((16,), jnp.int32)

    def _vpu_transpose(l, b):
        wordbases = []
        for k in range(8):
            v = idxv[l, pl.ds(16 * k, 16)]
            wordbases.append((it16 + 16 * k) * 128 + (v & 1) * 64)

        @plsc.parallel_loop(0, _EMB, unroll=8)
        def cbody(c):
            for k in range(8):
                val = plsc.load_gather(gbuf.at[b], [zero16, wordbases[k] + c])
                tbuf[b, c, pl.ds(16 * k, 16)] = val

    for b in range(4):
        _make_pidx(b, b)
        _start_gather(b)

    def body(u, carry):
        l0 = 4 * u
        for b in range(4):
            _wait_gather(b)

            @pl.when(u > 0)
            def _(b=b):
                _wait_out(b)

            _vpu_transpose(l0 + b, b)
            _start_out(l0 + b, b)

            @pl.when(u + 1 < _L // 4)
            def _(l=l0 + b, b=b):
                _make_pidx(l + 4, b)
                _start_gather(b)
        return carry

    lax.fori_loop(0, _L // 4, body, 0)
    for b in range(4):
        _wait_out(b)


def kernel(sequences, token_embedding_matrix):
    tT = jnp.transpose(token_embedding_matrix)
    tail = jnp.reshape(token_embedding_matrix[_NBLK * 128:, :], (32, 128))
    tlin = _transpose_table(tT, tail)
    seqT = jnp.transpose(sequences.astype(jnp.int32))
    outT = _emb_gather(seqT, tlin)
    emb = jnp.transpose(outT, (2, 0, 1))
    return emb, _L


# speedup vs baseline: 1.2160x; 1.2160x over previous
import functools

import jax
import jax.numpy as jnp
from jax import lax
from jax.experimental import pallas as pl
from jax.experimental.pallas import tpu as pltpu
from jax.experimental.pallas import tpu_sc as plsc

_B = 4096
_L = 200
_EMB = 64
_V = 1000000
_NW = 32
_VP = _V // 2
_NBLK = (_V // 128)
_BPW = _NBLK // _NW
_REM = _NBLK - _BPW * _NW
_PAIRS_A = _BPW // 2
_BB = _B // _NW
_PAIRS_B = _L // 2

_mesh = plsc.VectorSubcoreMesh(core_axis_name="c", subcore_axis_name="s")
_params = pltpu.CompilerParams(
    use_tc_tiling_on_sc=True,
    needs_layout_passes=False,
    disable_bounds_checks=True,
)


def _iota16():
    return lax.iota(jnp.int32, 16)


@functools.partial(
    pl.kernel,
    mesh=_mesh,
    out_type=jax.ShapeDtypeStruct((_VP, 128), jnp.float32),
    scratch_types=[
        pltpu.VMEM((4, _EMB, 128), jnp.float32),
        pltpu.VMEM((4, _EMB, 128), jnp.float32),
        pltpu.VMEM((32, 128), jnp.float32),
        pltpu.SemaphoreType.DMA,
        pltpu.SemaphoreType.DMA,
    ],
    compiler_params=_params,
)
def _transpose_table(tT_hbm, tail_hbm, tlin_hbm, inv, obuf, tailv, isem, osem):
    wid = lax.axis_index("s") * 2 + lax.axis_index("c")
    base = wid * _BPW
    it16 = _iota16()

    def _start_in(j, b):
        pltpu.async_copy(tT_hbm.at[:, pl.ds(j * 128, 128)], inv.at[b], isem)

    def _wait_in(b):
        pltpu.make_async_copy(tT_hbm.at[:, pl.ds(0, 128)], inv.at[b], isem).wait()

    def _start_out(j, b):
        pltpu.async_copy(obuf.at[b], tlin_hbm.at[pl.ds(j * 64, 64)], osem)

    def _wait_out(b):
        pltpu.make_async_copy(obuf.at[b], tlin_hbm.at[pl.ds(0, 64)], osem).wait()

    zero16 = jnp.zeros((16,), jnp.int32)
    sbases = [((it16 + 16 * k) >> 1) * 128 + ((it16 + 16 * k) & 1) * 64
              for k in range(8)]

    def _vpu_transpose(b):
        @plsc.parallel_loop(0, 64, unroll=8)
        def cbody(c):
            for k in range(8):
                val = inv[b, c, pl.ds(16 * k, 16)]
                plsc.store_scatter(obuf.at[b], [zero16, sbases[k] + c], val)

    for b in range(4):
        _start_in(base + b, b)

    def body(u, carry):
        t0 = base + 4 * u
        for b in range(4):
            _wait_in(b)

            @pl.when(u > 0)
            def _(b=b):
                _wait_out(b)

            _vpu_transpose(b)
            _start_out(t0 + b, b)

            @pl.when(u + 1 < _BPW // 4)
            def _(t=t0 + b, b=b):
                _start_in(t + 4, b)
        return carry

    lax.fori_loop(0, _BPW // 4, body, 0)
    for b in range(4):
        _wait_out(b)

    @pl.when(wid < _REM)
    def _():
        jx = _NBLK - _REM + wid
        _start_in(jx, 0)
        _wait_in(0)
        _vpu_transpose(0)
        _start_out(jx, 0)
        _wait_out(0)

    @pl.when(wid == _NW - 1)
    def _():
        pltpu.sync_copy(tail_hbm, tailv)
        pltpu.sync_copy(tailv, tlin_hbm.at[pl.ds(_NBLK * 64, 32)])


@functools.partial(
    pl.kernel,
    mesh=_mesh,
    out_type=jax.ShapeDtypeStruct((_L, _EMB, _B), jnp.float32),
    scratch_types=[
        pltpu.VMEM((_L, 128), jnp.int32),
        pltpu.VMEM((4, 128), jnp.int32),
        pltpu.VMEM((4, 128, 128), jnp.float32),
        pltpu.VMEM((4, _EMB, 128), jnp.float32),
        pltpu.SemaphoreType.DMA,
        pltpu.SemaphoreType.DMA,
    ],
    compiler_params=_params,
)
def _emb_gather(seqT_hbm, tlin_hbm, out_hbm, idxv, pidx, gbuf, tbuf, gsem, osem):
    wid = lax.axis_index("s") * 2 + lax.axis_index("c")
    bcol = wid * _BB
    it16 = _iota16()

    pltpu.sync_copy(seqT_hbm.at[:, pl.ds(bcol, _BB)], idxv)

    def _make_pidx(l, b):
        for k in range(8):
            v = idxv[l, pl.ds(16 * k, 16)]
            pidx[b, pl.ds(16 * k, 16)] = lax.shift_right_logical(v, 1)

    def _start_gather(b):
        pltpu.async_copy(tlin_hbm.at[pidx.at[b]], gbuf.at[b], gsem)

    def _wait_gather(b):
        pltpu.make_async_copy(tlin_hbm.at[pidx.at[b]], gbuf.at[b], gsem).wait()

    def _start_out(l, b):
        pltpu.async_copy(tbuf.at[b], out_hbm.at[l, :, pl.ds(bcol, _BB)], osem)

    def _wait_out(b):
        pltpu.make_async_copy(tbuf.at[b], out_hbm.at[0, :, pl.ds(bcol, _BB)], osem).wait()

    zero16 = jnp.zeros((16,), jnp.int32)

    def _vpu_transpose(l, b):
        wordbases = []
        for k in range(8):
            v = idxv[l, pl.ds(16 * k, 16)]
            wordbases.append((it16 + 16 * k) * 128 + (v & 1) * 64)

        @plsc.parallel_loop(0, _EMB, unroll=8)
        def cbody(c):
            for k in range(8):
                val = plsc.load_gather(gbuf.at[b], [zero16, wordbases[k] + c])
                tbuf[b, c, pl.ds(16 * k, 16)] = val

    for b in range(4):
        _make_pidx(b, b)
        _start_gather(b)

    def body(u, carry):
        l0 = 4 * u
        for b in range(4):
            _wait_gather(b)

            @pl.when(u > 0)
            def _(b=b):
                _wait_out(b)

            _vpu_transpose(l0 + b, b)
            _start_out(l0 + b, b)

            @pl.when(u + 1 < _L // 4)
            def _(l=l0 + b, b=b):
                _make_pidx(l + 4, b)
                _start_gather(b)
        return carry

    lax.fori_loop(0, _L // 4, body, 0)
    for b in range(4):
        _wait_out(b)


def kernel(sequences, token_embedding_matrix):
    tlin = jnp.reshape(token_embedding_matrix, (_VP, 128))
    seqT = jnp.transpose(sequences.astype(jnp.int32))
    outT = _emb_gather(seqT, tlin)
    emb = jnp.transpose(outT, (2, 0, 1))
    return emb, _L


# speedup vs baseline: 1.2386x; 1.0186x over previous
import functools

import jax
import jax.numpy as jnp
from jax import lax
from jax.experimental import pallas as pl
from jax.experimental.pallas import tpu as pltpu
from jax.experimental.pallas import tpu_sc as plsc

_B = 4096
_L = 200
_EMB = 64
_TOTAL = _B * _L
_NW = 32
_PER_W = _TOTAL // _NW
_CHUNK = 800
_STEPS = _PER_W // _CHUNK
_PAIRS = _STEPS // 2

_mesh = plsc.VectorSubcoreMesh(core_axis_name="c", subcore_axis_name="s")


@functools.partial(
    pl.kernel,
    mesh=_mesh,
    out_type=jax.ShapeDtypeStruct((_TOTAL, _EMB), jnp.float32),
    scratch_types=[
        pltpu.VMEM((_STEPS, _CHUNK), jnp.int32),
        pltpu.VMEM((2, _CHUNK, _EMB), jnp.float32),
        pltpu.SemaphoreType.DMA,
    ],
    compiler_params=pltpu.CompilerParams(use_tc_tiling_on_sc=False),
)
def _emb_gather(idx_hbm, table_hbm, out_hbm, idx_v, rows_v, gsem):
    wid = lax.axis_index("s") * 2 + lax.axis_index("c")
    base = wid * _PER_W

    pltpu.sync_copy(idx_hbm.at[wid], idx_v)

    def _start(i, b):
        pltpu.async_copy(table_hbm.at[idx_v.at[i]], rows_v.at[b], gsem)

    def _finish(i, b):
        pltpu.make_async_copy(table_hbm.at[idx_v.at[i]], rows_v.at[b], gsem).wait()
        off = pl.multiple_of(base + i * _CHUNK, 8)
        pltpu.sync_copy(rows_v.at[b], out_hbm.at[pl.ds(off, _CHUNK)])

    _start(0, 0)

    def body(j, carry):
        i0 = 2 * j
        _start(i0 + 1, 1)
        _finish(i0, 0)

        @pl.when(j + 1 < _PAIRS)
        def _():
            _start(i0 + 2, 0)

        _finish(i0 + 1, 1)
        return carry

    lax.fori_loop(0, _PAIRS, body, 0)


def kernel(sequences, token_embedding_matrix):
    idx = sequences.reshape(_NW, _STEPS, _CHUNK).astype(jnp.int32)
    flat = _emb_gather(idx, token_embedding_matrix)
    return flat.reshape(_B, _L, _EMB), _L
